# Initial kernel scaffold; baseline (speedup 1.0000x reference)
import jax
import jax.numpy as jnp
from jax.experimental import pallas as pl


def kernel(x, embed1_w, embed2_w, bias):
    raise NotImplementedError("write your pallas kernel here")


# speedup vs baseline: 1.3382x; 1.3382x over previous
import functools

import jax
import jax.numpy as jnp
import numpy as np
from jax import lax
from jax.experimental import pallas as pl
from jax.experimental.pallas import tpu as pltpu
from jax.experimental.pallas import tpu_sc as plsc

_FIELD_DIM = 38461
_F = 26
_D = 16
_B = 16384
_TOTAL = _FIELD_DIM * _F

_NC, _NS = 2, 16
_NW = _NC * _NS
_ROWS_W = _B // _NW
_CHUNK_ROWS = 64
_NCHUNK = _ROWS_W // _CHUNK_ROWS
_NIDX = _CHUNK_ROWS * _F
_STREAM = 128
_NSTREAM = _NIDX // _STREAM
_OFFP = 208


def _fm_body(x_hbm, off_hbm, bias_hbm, e1_hbm, e2_hbm, out_hbm,
             idx0, idx1, e20, e21, e10, e11,
             off_v, bias_v, t_v, out_v,
             se2_0, se2_1, se1_0, se1_1):
    idx_v = (idx0, idx1)
    e2_v = (e20, e21)
    e1_v = (e10, e11)
    sem_e2 = (se2_0, se2_1)
    sem_e1 = (se1_0, se1_1)

    wid = lax.axis_index("s") * _NC + lax.axis_index("c")
    iota = lax.iota(jnp.int32, _D)

    pltpu.sync_copy(off_hbm, off_v)
    pltpu.sync_copy(bias_hbm, bias_v)

    def fire(s, c):
        base = wid * (_NCHUNK * _NIDX) + c * _NIDX
        pltpu.sync_copy(x_hbm.at[pl.ds(base, _NIDX)], idx_v[s])
        for i in range(_NIDX // _D):
            o = (i * _D) % _OFFP
            sl = pl.ds(i * _D, _D)
            idx_v[s][sl] = idx_v[s][sl] + off_v[pl.ds(o, _D)]
        for j in range(_NSTREAM):
            pltpu.make_async_copy(
                e2_hbm.at[idx_v[s].at[pl.ds(j * _STREAM, _STREAM)]],
                e2_v[s].at[pl.ds(j * _STREAM, _STREAM), :],
                sem_e2[s]).start()
            pltpu.make_async_copy(
                e1_hbm.at[idx_v[s].at[pl.ds(j * _STREAM, _STREAM)]],
                e1_v[s].at[pl.ds(j * _STREAM, _STREAM)],
                sem_e1[s]).start()

    def wait_all(s):
        pltpu.make_async_copy(
            e2_hbm.at[pl.ds(0, _NIDX), :], e2_v[s], sem_e2[s]).wait()
        pltpu.make_async_copy(
            e1_hbm.at[pl.ds(0, _NIDX)], e1_v[s], sem_e1[s]).wait()

    def compute(s, c):
        zeros = jnp.zeros((_D,), jnp.float32)

        def row_body(r, _):
            acc = zeros
            acc2 = zeros
            rbase = r * _F
            for f in range(_F):
                rv = jnp.full((_D,), rbase + f, jnp.int32)
                v = plsc.load_gather(e2_v[s], [rv, iota])
                acc = acc + v
                acc2 = acc2 + v * v
            t = acc * acc - acc2
            plsc.store_scatter(t_v, [jnp.full((_D,), r, jnp.int32), iota], t)
            return 0

        lax.fori_loop(0, _CHUNK_ROWS, row_body, 0)

        bias_vec = bias_v[...]
        for g in range(_CHUNK_ROWS // _D):
            rows16 = iota + g * _D
            inter = zeros
            for d in range(_D):
                inter = inter + plsc.load_gather(
                    t_v, [rows16, jnp.full((_D,), d, jnp.int32)])
            nbase = rows16 * _F
            lin = zeros
            for f in range(_F):
                lin = lin + plsc.load_gather(e1_v[s], [nbase + f])
            z = lin + bias_vec + 0.5 * inter
            sig = 1.0 / (1.0 + jnp.exp(-z))
            plsc.store_scatter(out_v, [rows16 + c * _CHUNK_ROWS], sig)

    fire(0, 0)

    def loop_body(t, _):
        c0 = 2 * t
        fire(1, c0 + 1)
        wait_all(0)
        compute(0, c0)

        @pl.when(t < _NCHUNK // 2 - 1)
        def _():
            fire(0, c0 + 2)

        wait_all(1)
        compute(1, c0 + 1)
        return 0

    lax.fori_loop(0, _NCHUNK // 2, loop_body, 0)

    pltpu.sync_copy(out_v, out_hbm.at[pl.ds(wid * _ROWS_W, _ROWS_W)])


@jax.jit
def kernel(x, embed1_w, embed2_w, bias):
    x_flat = x.reshape(_B * _F)
    offs = jnp.asarray(
        (np.arange(_OFFP, dtype=np.int64) % _F) * _FIELD_DIM, dtype=jnp.int32)
    bias16 = jnp.broadcast_to(bias.astype(jnp.float32), (_D,))
    e1_flat = embed1_w.reshape(_TOTAL)

    mesh = plsc.VectorSubcoreMesh(core_axis_name="c", subcore_axis_name="s")
    fm = pl.kernel(
        _fm_body,
        out_type=jax.ShapeDtypeStruct((_B,), jnp.float32),
        mesh=mesh,
        compiler_params=pltpu.CompilerParams(
            use_tc_tiling_on_sc=False, needs_layout_passes=False),
        scratch_types=[
            pltpu.VMEM((_NIDX,), jnp.int32),
            pltpu.VMEM((_NIDX,), jnp.int32),
            pltpu.VMEM((_NIDX, _D), jnp.float32),
            pltpu.VMEM((_NIDX, _D), jnp.float32),
            pltpu.VMEM((_NIDX,), jnp.float32),
            pltpu.VMEM((_NIDX,), jnp.float32),
            pltpu.VMEM((_OFFP,), jnp.int32),
            pltpu.VMEM((_D,), jnp.float32),
            pltpu.VMEM((_CHUNK_ROWS, _D), jnp.float32),
            pltpu.VMEM((_ROWS_W,), jnp.float32),
            pltpu.SemaphoreType.DMA,
            pltpu.SemaphoreType.DMA,
            pltpu.SemaphoreType.DMA,
            pltpu.SemaphoreType.DMA,
        ],
    )
    return fm(x_flat, offs, bias16, e1_flat, embed2_w)


# speedup vs baseline: 1.3396x; 1.0010x over previous
import functools

import jax
import jax.numpy as jnp
import numpy as np
from jax import lax
from jax.experimental import pallas as pl
from jax.experimental.pallas import tpu as pltpu
from jax.experimental.pallas import tpu_sc as plsc

_FIELD_DIM = 38461
_F = 26
_D = 16
_B = 16384
_TOTAL = _FIELD_DIM * _F

_NC, _NS = 2, 16
_NW = _NC * _NS
_ROWS_W = _B // _NW
_CHUNK_ROWS = 64
_NCHUNK = _ROWS_W // _CHUNK_ROWS
_NIDX = _CHUNK_ROWS * _F
_STREAM = 128
_NSTREAM = _NIDX // _STREAM
_OFFP = 208


def _fm_body(x_hbm, off_hbm, bias_hbm, e1_hbm, e2_hbm, out_hbm,
             idx0, idx1, e20, e21, e10, e11,
             off_v, bias_v, t_v, out_v,
             se2_0, se2_1, se1_0, se1_1):
    idx_v = (idx0, idx1)
    e2_v = (e20, e21)
    e1_v = (e10, e11)
    sem_e2 = (se2_0, se2_1)
    sem_e1 = (se1_0, se1_1)

    wid = lax.axis_index("s") * _NC + lax.axis_index("c")
    iota = lax.iota(jnp.int32, _D)

    pltpu.sync_copy(off_hbm, off_v)
    pltpu.sync_copy(bias_hbm, bias_v)

    def fire(s, c):
        base = wid * (_NCHUNK * _NIDX) + c * _NIDX
        pltpu.sync_copy(x_hbm.at[pl.ds(base, _NIDX)], idx_v[s])
        for i in range(_NIDX // _D):
            o = (i * _D) % _OFFP
            sl = pl.ds(i * _D, _D)
            idx_v[s][sl] = idx_v[s][sl] + off_v[pl.ds(o, _D)]
        for j in range(_NSTREAM):
            pltpu.make_async_copy(
                e2_hbm.at[idx_v[s].at[pl.ds(j * _STREAM, _STREAM)]],
                e2_v[s].at[pl.ds(j * _STREAM, _STREAM), :],
                sem_e2[s]).start()
            pltpu.make_async_copy(
                e1_hbm.at[idx_v[s].at[pl.ds(j * _STREAM, _STREAM)]],
                e1_v[s].at[pl.ds(j * _STREAM, _STREAM)],
                sem_e1[s]).start()

    def wait_all(s):
        pltpu.make_async_copy(
            e2_hbm.at[pl.ds(0, _NIDX), :], e2_v[s], sem_e2[s]).wait()
        pltpu.make_async_copy(
            e1_hbm.at[pl.ds(0, _NIDX)], e1_v[s], sem_e1[s]).wait()

    def compute(s, c):
        zeros = jnp.zeros((_D,), jnp.float32)

        def row_body(r, _):
            acc = zeros
            acc2 = zeros
            rbase = r * _F
            for f in range(_F):
                v = e2_v[s][rbase + f, :]
                acc = acc + v
                acc2 = acc2 + v * v
            t = acc * acc - acc2
            plsc.store_scatter(t_v, [jnp.full((_D,), r, jnp.int32), iota], t)
            return 0

        lax.fori_loop(0, _CHUNK_ROWS, row_body, 0)

        bias_vec = bias_v[...]
        for g in range(_CHUNK_ROWS // _D):
            rows16 = iota + g * _D
            inter = zeros
            for d in range(_D):
                inter = inter + plsc.load_gather(
                    t_v, [rows16, jnp.full((_D,), d, jnp.int32)])
            nbase = rows16 * _F
            lin = zeros
            for f in range(_F):
                lin = lin + plsc.load_gather(e1_v[s], [nbase + f])
            z = lin + bias_vec + 0.5 * inter
            sig = 1.0 / (1.0 + jnp.exp(-z))
            plsc.store_scatter(out_v, [rows16 + c * _CHUNK_ROWS], sig)

    fire(0, 0)

    def loop_body(t, _):
        c0 = 2 * t
        fire(1, c0 + 1)
        wait_all(0)
        compute(0, c0)

        @pl.when(t < _NCHUNK // 2 - 1)
        def _():
            fire(0, c0 + 2)

        wait_all(1)
        compute(1, c0 + 1)
        return 0

    lax.fori_loop(0, _NCHUNK // 2, loop_body, 0)

    pltpu.sync_copy(out_v, out_hbm.at[pl.ds(wid * _ROWS_W, _ROWS_W)])


@jax.jit
def kernel(x, embed1_w, embed2_w, bias):
    x_flat = x.reshape(_B * _F)
    offs = jnp.asarray(
        (np.arange(_OFFP, dtype=np.int64) % _F) * _FIELD_DIM, dtype=jnp.int32)
    bias16 = jnp.broadcast_to(bias.astype(jnp.float32), (_D,))
    e1_flat = embed1_w.reshape(_TOTAL)

    mesh = plsc.VectorSubcoreMesh(core_axis_name="c", subcore_axis_name="s")
    fm = pl.kernel(
        _fm_body,
        out_type=jax.ShapeDtypeStruct((_B,), jnp.float32),
        mesh=mesh,
        compiler_params=pltpu.CompilerParams(
            use_tc_tiling_on_sc=False, needs_layout_passes=False),
        scratch_types=[
            pltpu.VMEM((_NIDX,), jnp.int32),
            pltpu.VMEM((_NIDX,), jnp.int32),
            pltpu.VMEM((_NIDX, _D), jnp.float32),
            pltpu.VMEM((_NIDX, _D), jnp.float32),
            pltpu.VMEM((_NIDX,), jnp.float32),
            pltpu.VMEM((_NIDX,), jnp.float32),
            pltpu.VMEM((_OFFP,), jnp.int32),
            pltpu.VMEM((_D,), jnp.float32),
            pltpu.VMEM((_CHUNK_ROWS, _D), jnp.float32),
            pltpu.VMEM((_ROWS_W,), jnp.float32),
            pltpu.SemaphoreType.DMA,
            pltpu.SemaphoreType.DMA,
            pltpu.SemaphoreType.DMA,
            pltpu.SemaphoreType.DMA,
        ],
    )
    return fm(x_flat, offs, bias16, e1_flat, embed2_w)


# speedup vs baseline: 5.1892x; 3.8737x over previous
import jax
import jax.numpy as jnp
import numpy as np
from jax import lax
from jax.experimental import pallas as pl
from jax.experimental.pallas import tpu as pltpu
from jax.experimental.pallas import tpu_sc as plsc

_FIELD_DIM = 38461
_F = 26
_D = 16
_B = 16384
_TOTAL = _FIELD_DIM * _F
_TILEC = 7813
_TOTALP = _TILEC * 128

_NC, _NS = 2, 16
_NW = _NC * _NS

_BLK = 128
_GRP = 4
_GCOLS = _GRP * _BLK
_BPW = 244
_NGRP = _BPW // _GRP

_ROWS_W = _B // _NW
_CHUNK_ROWS = 64
_NCHUNK = _ROWS_W // _CHUNK_ROWS
_NIDX = _CHUNK_ROWS * _F
_STREAM = 128
_NSTREAM = _NIDX // _STREAM
_OFFP = 208


def _detile_body(e2t_hbm, tail_hbm, out_hbm,
                 in0, in1, t0, t1, si0, si1, so0, so1):
    in_v = (in0, in1)
    out_v = (t0, t1)
    sem_in = (si0, si1)
    sem_out = (so0, so1)
    wid = lax.axis_index("s") * _NC + lax.axis_index("c")
    iota = lax.iota(jnp.int32, _D)
    c0 = wid * _BPW

    def fire_in(s, g):
        pltpu.make_async_copy(
            e2t_hbm.at[:, pl.ds((c0 + g * _GRP) * _BLK, _GCOLS)],
            in_v[s], sem_in[s]).start()

    def wait_in(s):
        pltpu.make_async_copy(
            e2t_hbm.at[:, pl.ds(0, _GCOLS)], in_v[s], sem_in[s]).wait()

    iot16 = iota * _D

    def transpose(s, ncols):
        def tb(lg, _):
            for half in range(2):
                l0 = (lg * 2 + half) * _D
                v_idx = jnp.full((_D,), l0 * _D, jnp.int32) + iot16
                vals = [in_v[s][d, pl.ds(l0, _D)] for d in range(_D)]
                for d in range(_D):
                    plsc.store_scatter(out_v[s], [v_idx + d], vals[d])
            return 0

        lax.fori_loop(0, ncols // (2 * _D), tb, 0)

    def fire_out(s, g):
        pltpu.make_async_copy(
            out_v[s], out_hbm.at[pl.ds((c0 + g * _GRP) * _BLK * _D, _GCOLS * _D)],
            sem_out[s]).start()

    def wait_out(s):
        pltpu.make_async_copy(
            out_hbm.at[pl.ds(0, _GCOLS * _D)], out_v[s], sem_out[s]).wait()

    fire_in(0, 0)

    def body(t, _):
        g0 = 2 * t
        g1 = g0 + 1

        @pl.when(g1 < _NGRP)
        def _():
            fire_in(1, g1)

        wait_in(0)

        @pl.when(t > 0)
        def _():
            wait_out(0)

        transpose(0, _GCOLS)
        fire_out(0, g0)

        @pl.when(g0 + 2 < _NGRP)
        def _():
            fire_in(0, g0 + 2)

        @pl.when(g1 < _NGRP)
        def _():
            wait_in(1)

            @pl.when(t > 0)
            def _():
                wait_out(1)

            transpose(1, _GCOLS)
            fire_out(1, g1)
        return 0

    lax.fori_loop(0, (_NGRP + 1) // 2, body, 0)
    wait_out(0)
    wait_out(1)

    tail_blk = 32 * _BPW + wid

    @pl.when(wid < 4)
    def _():
        pltpu.make_async_copy(
            e2t_hbm.at[:, pl.ds(tail_blk * _BLK, _BLK)],
            in_v[0].at[:, pl.ds(0, _BLK)], sem_in[0]).start()
        pltpu.make_async_copy(
            e2t_hbm.at[:, pl.ds(0, _BLK)], in_v[0].at[:, pl.ds(0, _BLK)],
            sem_in[0]).wait()
        transpose(0, _BLK)
        pltpu.make_async_copy(
            out_v[0].at[pl.ds(0, _BLK * _D)],
            out_hbm.at[pl.ds(tail_blk * _BLK * _D, _BLK * _D)],
            sem_out[0]).start()
        pltpu.make_async_copy(
            out_hbm.at[pl.ds(0, _BLK * _D)],
            out_v[0].at[pl.ds(0, _BLK * _D)], sem_out[0]).wait()

    @pl.when(wid == 4)
    def _():
        pltpu.make_async_copy(tail_hbm, in_v[0].at[:, pl.ds(0, _BLK)],
                              sem_in[0]).start()
        pltpu.make_async_copy(tail_hbm, in_v[0].at[:, pl.ds(0, _BLK)],
                              sem_in[0]).wait()
        transpose(0, _BLK)
        pltpu.make_async_copy(
            out_v[0].at[pl.ds(0, _BLK * _D)],
            out_hbm.at[pl.ds((_TILEC - 1) * _BLK * _D, _BLK * _D)],
            sem_out[0]).start()
        pltpu.make_async_copy(
            out_hbm.at[pl.ds(0, _BLK * _D)],
            out_v[0].at[pl.ds(0, _BLK * _D)], sem_out[0]).wait()


def _fm_body(x_hbm, off_hbm, bias_hbm, e1_hbm, e2_hbm, out_hbm,
             idx0, idx1, e20, e21, e10, e11,
             off_v, bias_v, t_v, out_v,
             se2_0, se2_1, se1_0, se1_1):
    idx_v = (idx0, idx1)
    e2_v = (e20, e21)
    e1_v = (e10, e11)
    sem_e2 = (se2_0, se2_1)
    sem_e1 = (se1_0, se1_1)

    wid = lax.axis_index("s") * _NC + lax.axis_index("c")
    iota = lax.iota(jnp.int32, _D)

    pltpu.sync_copy(off_hbm, off_v)
    pltpu.sync_copy(bias_hbm, bias_v)

    def fire(s, c):
        base = wid * (_NCHUNK * _NIDX) + c * _NIDX
        pltpu.sync_copy(x_hbm.at[pl.ds(base, _NIDX)], idx_v[s])
        for i in range(_NIDX // _D):
            o = (i * _D) % _OFFP
            sl = pl.ds(i * _D, _D)
            idx_v[s][sl] = idx_v[s][sl] + off_v[pl.ds(o, _D)]
        for j in range(_NSTREAM):
            pltpu.make_async_copy(
                e2_hbm.at[idx_v[s].at[pl.ds(j * _STREAM, _STREAM)]],
                e2_v[s].at[pl.ds(j * _STREAM, _STREAM), :],
                sem_e2[s]).start()
            pltpu.make_async_copy(
                e1_hbm.at[idx_v[s].at[pl.ds(j * _STREAM, _STREAM)]],
                e1_v[s].at[pl.ds(j * _STREAM, _STREAM)],
                sem_e1[s]).start()

    def wait_all(s):
        pltpu.make_async_copy(
            e2_hbm.at[pl.ds(0, _NIDX), :], e2_v[s], sem_e2[s]).wait()
        pltpu.make_async_copy(
            e1_hbm.at[pl.ds(0, _NIDX)], e1_v[s], sem_e1[s]).wait()

    def compute(s, c):
        zeros = jnp.zeros((_D,), jnp.float32)

        def row_body(r, _):
            a0 = zeros
            a1 = zeros
            b0 = zeros
            b1 = zeros
            rbase = r * _F
            for f in range(0, _F, 2):
                v0 = e2_v[s][rbase + f, :]
                v1 = e2_v[s][rbase + f + 1, :]
                a0 = a0 + v0
                b0 = b0 + v0 * v0
                a1 = a1 + v1
                b1 = b1 + v1 * v1
            acc = a0 + a1
            acc2 = b0 + b1
            t = acc * acc - acc2
            plsc.store_scatter(t_v, [jnp.full((_D,), r, jnp.int32), iota], t)
            return 0

        lax.fori_loop(0, _CHUNK_ROWS, row_body, 0)

        bias_vec = bias_v[...]
        for g in range(_CHUNK_ROWS // _D):
            rows16 = iota + g * _D
            inter = zeros
            for d in range(_D):
                inter = inter + plsc.load_gather(
                    t_v, [rows16, jnp.full((_D,), d, jnp.int32)])
            nbase = rows16 * _F
            lin = zeros
            for f in range(_F):
                lin = lin + plsc.load_gather(e1_v[s], [nbase + f])
            z = lin + bias_vec + 0.5 * inter
            sig = 1.0 / (1.0 + jnp.exp(-z))
            plsc.store_scatter(out_v, [rows16 + c * _CHUNK_ROWS], sig)

    fire(0, 0)

    def loop_body(t, _):
        c0 = 2 * t
        fire(1, c0 + 1)
        wait_all(0)
        compute(0, c0)

        @pl.when(t < _NCHUNK // 2 - 1)
        def _():
            fire(0, c0 + 2)

        wait_all(1)
        compute(1, c0 + 1)
        return 0

    lax.fori_loop(0, _NCHUNK // 2, loop_body, 0)

    pltpu.sync_copy(out_v, out_hbm.at[pl.ds(wid * _ROWS_W, _ROWS_W)])


@jax.jit
def kernel(x, embed1_w, embed2_w, bias):
    x_flat = x.reshape(_B * _F)
    offs = jnp.asarray(
        (np.arange(_OFFP, dtype=np.int64) % _F) * _FIELD_DIM, dtype=jnp.int32)
    bias16 = jnp.broadcast_to(bias.astype(jnp.float32), (_D,))
    e1_flat = embed1_w.reshape(_TOTAL)

    mesh = plsc.VectorSubcoreMesh(core_axis_name="c", subcore_axis_name="s")

    e2t = embed2_w.T
    tail_t = jnp.pad(embed2_w[(_TILEC - 1) * _BLK:],
                     ((0, _TOTALP - _TOTAL), (0, 0))).T
    detile = pl.kernel(
        _detile_body,
        out_type=jax.ShapeDtypeStruct((_TOTALP * _D,), jnp.float32),
        mesh=mesh,
        compiler_params=pltpu.CompilerParams(
            use_tc_tiling_on_sc=True, needs_layout_passes=False),
        scratch_types=[
            pltpu.VMEM((_D, _GCOLS), jnp.float32),
            pltpu.VMEM((_D, _GCOLS), jnp.float32),
            pltpu.VMEM((_GCOLS * _D,), jnp.float32),
            pltpu.VMEM((_GCOLS * _D,), jnp.float32),
            pltpu.SemaphoreType.DMA,
            pltpu.SemaphoreType.DMA,
            pltpu.SemaphoreType.DMA,
            pltpu.SemaphoreType.DMA,
        ],
    )
    e2m = detile(e2t, tail_t).reshape(_TOTALP, _D)

    fm = pl.kernel(
        _fm_body,
        out_type=jax.ShapeDtypeStruct((_B,), jnp.float32),
        mesh=mesh,
        compiler_params=pltpu.CompilerParams(
            use_tc_tiling_on_sc=False, needs_layout_passes=False),
        scratch_types=[
            pltpu.VMEM((_NIDX,), jnp.int32),
            pltpu.VMEM((_NIDX,), jnp.int32),
            pltpu.VMEM((_NIDX, _D), jnp.float32),
            pltpu.VMEM((_NIDX, _D), jnp.float32),
            pltpu.VMEM((_NIDX,), jnp.float32),
            pltpu.VMEM((_NIDX,), jnp.float32),
            pltpu.VMEM((_OFFP,), jnp.int32),
            pltpu.VMEM((_D,), jnp.float32),
            pltpu.VMEM((_CHUNK_ROWS, _D), jnp.float32),
            pltpu.VMEM((_ROWS_W,), jnp.float32),
            pltpu.SemaphoreType.DMA,
            pltpu.SemaphoreType.DMA,
            pltpu.SemaphoreType.DMA,
            pltpu.SemaphoreType.DMA,
        ],
    )
    return fm(x_flat, offs, bias16, e1_flat, e2m)
